# baseline (speedup 1.0000x reference)
import functools

import jax
import jax.numpy as jnp
from jax import lax
from jax.experimental import pallas as pl
from jax.experimental.pallas import tpu as pltpu
from jax.experimental.pallas import tpu_sc as plsc

VOCAB = 100000
EMBED_DIM = 64
BATCH = 4096

_info = plsc.get_sparse_core_info()
_NC = _info.num_cores
_NS = _info.num_subcores
_L = _info.num_lanes
_NW = _NC * _NS
_BPW = BATCH // _NW
_GROUPS = _BPW // _L

_mesh = plsc.VectorSubcoreMesh(core_axis_name="c", subcore_axis_name="s")


@functools.partial(
    pl.kernel,
    mesh=_mesh,
    out_type=jax.ShapeDtypeStruct((_NW * _L,), jnp.float32),
    compiler_params=pltpu.CompilerParams(use_tc_tiling_on_sc=True,
                                         needs_layout_passes=False),
    scratch_types=[
        pltpu.VMEM((_BPW,), jnp.int32),
        pltpu.VMEM((_BPW,), jnp.int32),
        pltpu.VMEM((_BPW,), jnp.int32),
        pltpu.VMEM((_BPW,), jnp.int32),
        pltpu.VMEM((_BPW, 2 * EMBED_DIM), jnp.float32),
        pltpu.VMEM((_BPW, 2 * EMBED_DIM), jnp.float32),
        pltpu.VMEM((_L,), jnp.float32),
        pltpu.SemaphoreType.DMA,
        pltpu.SemaphoreType.DMA,
    ],
)
def _sc_partial_sumsq(table_hbm, head_hbm, tail_hbm, out_hbm,
                      hidx_v, tidx_v, hrow_id_v, trow_id_v,
                      hrow_v, trow_v, acc_v, sem_h, sem_t):
    wid = lax.axis_index("s") * _NC + lax.axis_index("c")
    base = wid * _BPW
    pltpu.sync_copy(head_hbm.at[pl.ds(base, _BPW)], hidx_v)
    pltpu.sync_copy(tail_hbm.at[pl.ds(base, _BPW)], tidx_v)

    for j in range(_BPW // _L):
        sl = pl.ds(j * _L, _L)
        hrow_id_v[sl] = lax.shift_right_logical(hidx_v[sl], 1)
        trow_id_v[sl] = lax.shift_right_logical(tidx_v[sl], 1)

    ch = pltpu.async_copy(table_hbm.at[hrow_id_v], hrow_v, sem_h)
    ct = pltpu.async_copy(table_hbm.at[trow_id_v], trow_v, sem_t)
    ch.wait()
    ct.wait()

    iota = lax.iota(jnp.int32, _L)
    total = jnp.zeros((_L,), jnp.float32)
    for g in range(_GROUPS):
        sl = pl.ds(g * _L, _L)
        rows = iota + (g * _L)
        hcol0 = (hidx_v[sl] & 1) * EMBED_DIM
        tcol0 = (tidx_v[sl] & 1) * EMBED_DIM

        def body(d4, accs):
            a0, a1, a2, a3 = accs
            outs = []
            for u in range(4):
                d = d4 * 4 + u
                h = plsc.load_gather(hrow_v, [rows, hcol0 + d])
                t = plsc.load_gather(trow_v, [rows, tcol0 + d])
                diff = h - t
                outs.append(accs[u] + diff * diff)
            return tuple(outs)

        zero = jnp.zeros((_L,), jnp.float32)
        a0, a1, a2, a3 = lax.fori_loop(0, EMBED_DIM // 4, body,
                                       (zero, zero, zero, zero))
        total = total + a0 + a1 + a2 + a3

    acc_v[...] = total
    pltpu.sync_copy(acc_v, out_hbm.at[pl.ds(wid * _L, _L)])


def kernel(table, head, tail):
    table2 = jnp.reshape(table, (VOCAB // 2, 2 * EMBED_DIM))
    partials = _sc_partial_sumsq(
        table2, head.astype(jnp.int32), tail.astype(jnp.int32))
    return jnp.sqrt(jnp.sum(partials))


# speedup vs baseline: 1.6181x; 1.6181x over previous
import functools

import jax
import jax.numpy as jnp
from jax import lax
from jax.experimental import pallas as pl
from jax.experimental.pallas import tpu as pltpu
from jax.experimental.pallas import tpu_sc as plsc

VOCAB = 100000
EMBED_DIM = 64
BATCH = 4096

_info = plsc.get_sparse_core_info()
_NC = _info.num_cores
_NS = _info.num_subcores
_L = _info.num_lanes
_NW = _NC * _NS
_BPW = BATCH // _NW
_GROUPS = _BPW // _L
_CHUNKS = EMBED_DIM // _L

_mesh = plsc.VectorSubcoreMesh(core_axis_name="c", subcore_axis_name="s")


@functools.partial(
    pl.kernel,
    mesh=_mesh,
    out_type=jax.ShapeDtypeStruct((_NW * _L,), jnp.float32),
    compiler_params=pltpu.CompilerParams(use_tc_tiling_on_sc=True,
                                         needs_layout_passes=False),
    scratch_types=[
        pltpu.VMEM((_BPW,), jnp.int32),
        pltpu.VMEM((_BPW,), jnp.int32),
        pltpu.VMEM((_BPW, EMBED_DIM), jnp.float32),
        pltpu.VMEM((_BPW, EMBED_DIM), jnp.float32),
        pltpu.VMEM((_L,), jnp.float32),
        pltpu.SemaphoreType.DMA,
        pltpu.SemaphoreType.DMA,
    ],
)
def _sc_partial_sumsq(table_hbm, head_hbm, tail_hbm, out_hbm,
                      hidx_v, tidx_v, hrow_v, trow_v, acc_v, sem_h, sem_t):
    wid = lax.axis_index("s") * _NC + lax.axis_index("c")
    base = wid * _BPW
    pltpu.sync_copy(head_hbm.at[pl.ds(base, _BPW)], hidx_v)
    pltpu.sync_copy(tail_hbm.at[pl.ds(base, _BPW)], tidx_v)

    def issue(g, carry):
        hv = hidx_v[pl.ds(g * _L, _L)]
        tv = tidx_v[pl.ds(g * _L, _L)]
        for l in range(_L):
            r = g * _L + l
            pltpu.async_copy(table_hbm.at[pl.ds(hv[l], 1)],
                             hrow_v.at[pl.ds(r, 1)], sem_h)
            pltpu.async_copy(table_hbm.at[pl.ds(tv[l], 1)],
                             trow_v.at[pl.ds(r, 1)], sem_t)
        return carry

    lax.fori_loop(0, _GROUPS, issue, 0)

    pltpu.make_async_copy(table_hbm.at[pl.ds(0, _BPW)], hrow_v, sem_h).wait()
    pltpu.make_async_copy(table_hbm.at[pl.ds(0, _BPW)], trow_v, sem_t).wait()

    def body(r, accs):
        new = []
        for c in range(_CHUNKS):
            h = hrow_v[r, pl.ds(c * _L, _L)]
            t = trow_v[r, pl.ds(c * _L, _L)]
            d = h - t
            new.append(accs[c] + d * d)
        return tuple(new)

    zero = jnp.zeros((_L,), jnp.float32)
    accs = lax.fori_loop(0, _BPW, body, (zero,) * _CHUNKS)
    total = accs[0]
    for c in range(1, _CHUNKS):
        total = total + accs[c]
    acc_v[...] = total
    pltpu.sync_copy(acc_v, out_hbm.at[pl.ds(wid * _L, _L)])


def kernel(table, head, tail):
    partials = _sc_partial_sumsq(
        table, head.astype(jnp.int32), tail.astype(jnp.int32))
    return jnp.sqrt(jnp.sum(partials))
